# Initial kernel scaffold; baseline (speedup 1.0000x reference)
import jax
import jax.numpy as jnp
from jax.experimental import pallas as pl


def kernel(pos, edge_index, batch, W1, c1, W2, c2, W3, c3, W4, c4, g0, be0, g1, be1, g2, be2, g3, be3, g4, be4):
    raise NotImplementedError("write your pallas kernel here")


# speedup vs baseline: 11.4836x; 11.4836x over previous
import functools

import jax
import jax.numpy as jnp
from jax import lax
from jax.experimental import pallas as pl
from jax.experimental.pallas import tpu as pltpu
from jax.experimental.pallas import tpu_sc as plsc

N = 100000
E = 1600000
IN_CH = 13
HID = 32
G = 64
EPS = 1e-5

NC = 2
NS = 16
L = 16

NP = 100352
DUMP = 100000
EP = 1605632
RT = EP // 128

B_ROWS = RT // NS
B_K = 56
B_MC = B_ROWS // B_K
D_ROWS = RT // (NC * NS)
D_K = 56
D_MC = D_ROWS // D_K
ZCH = 784
ZN = NP // NS // ZCH
P_U = NP // NS // 128
GP = 72

@functools.cache
def _mesh():
    return plsc.VectorSubcoreMesh(core_axis_name="c", subcore_axis_name="s",
                                  num_cores=NC, num_subcores=NS)


def _fill(ref, rows, value):
    def body(i, _):
        ref[i] = jnp.full((L,), value, jnp.float32)
        return 0
    lax.fori_loop(0, rows, body, 0)


def _zero_acc(acc, zbuf, s):
    _fill(zbuf, ZCH, 0.0)
    t0 = s * (NP // NS)
    for k in range(ZN):
        pltpu.sync_copy(zbuf.at[pl.ds(0, ZCH)], acc.at[pl.ds(t0 + k * ZCH, ZCH)])


def _write_acc(acc, out, c, s):
    t0 = s * (NP // NS)
    for k in range(ZN):
        pltpu.sync_copy(acc.at[pl.ds(t0 + k * ZCH, ZCH)],
                        out.at[c, pl.ds(t0 + k * ZCH, ZCH)])


def _seg_body(h2, srcr, dstr, out, idx_s, idx_d, rowbuf, zbuf, acc, sem):
    c = lax.axis_index("c")
    s = lax.axis_index("s")
    _zero_acc(acc, zbuf, s)
    plsc.subcore_barrier()

    def macro(m, _):
        base = s * B_ROWS + m * B_K
        pltpu.sync_copy(srcr.at[pl.ds(base, B_K)], idx_s)
        pltpu.sync_copy(dstr.at[pl.ds(base, B_K)], idx_d)

        def inner(j, _):
            pltpu.async_copy(h2.at[c].at[idx_s.at[j]], rowbuf, sem).wait()
            pltpu.sync_copy(rowbuf, acc.at[idx_d.at[j]], add=True)
            return 0
        lax.fori_loop(0, B_K, inner, 0)
        return 0
    lax.fori_loop(0, B_MC, macro, 0)
    plsc.subcore_barrier()
    _write_acc(acc, out, c, s)


@functools.cache
def _seg_sum():
  return pl.kernel(
    _seg_body,
    out_type=jax.ShapeDtypeStruct((NC, NP, L), jnp.float32),
    mesh=_mesh(),
    compiler_params=pltpu.CompilerParams(use_tc_tiling_on_sc=False),
    scratch_types=[
        pltpu.VMEM((B_K, 128), jnp.int32),
        pltpu.VMEM((B_K, 128), jnp.int32),
        pltpu.VMEM((128, L), jnp.float32),
        pltpu.VMEM((ZCH, L), jnp.float32),
        pltpu.VMEM_SHARED((NP, L), jnp.float32),
        pltpu.SemaphoreType.DMA,
    ],
  )


def _fill1(ref, words, value):
    def body(i, _):
        ref[pl.ds(i * L, L)] = jnp.full((L,), value, jnp.float32)
        return 0
    lax.fori_loop(0, words // L, body, 0)


def _deg_body(dstr, out, idx_d, onesb, wbuf, ebuf, acc1):
    c = lax.axis_index("c")
    s = lax.axis_index("s")
    _fill1(wbuf, ZCH, 0.0)
    t0 = s * (NP // NS)
    for k in range(ZN):
        pltpu.sync_copy(wbuf.at[pl.ds(0, ZCH)], acc1.at[pl.ds(t0 + k * ZCH, ZCH)])
    _fill1(onesb, 128, 1.0)
    plsc.subcore_barrier()

    def macro(m, _):
        base = (c * NS + s) * D_ROWS + m * D_K
        pltpu.sync_copy(dstr.at[pl.ds(base, D_K)], idx_d)

        def inner(j, _):
            pltpu.sync_copy(onesb, acc1.at[idx_d.at[j]], add=True)
            return 0
        lax.fori_loop(0, D_K, inner, 0)
        return 0
    lax.fori_loop(0, D_MC, macro, 0)
    plsc.subcore_barrier()
    for k in range(ZN):
        r0 = t0 + k * ZCH
        pltpu.sync_copy(acc1.at[pl.ds(r0, ZCH)], wbuf)

        def expand(i, _):
            v = wbuf[pl.ds(i * L, L)]
            for kk in range(L):
                ebuf[i * L + kk] = jnp.broadcast_to(v[kk], (L,))
            return 0
        lax.fori_loop(0, ZCH // L, expand, 0)
        pltpu.sync_copy(ebuf, out.at[c, pl.ds(r0, ZCH)])


@functools.cache
def _deg_count():
  return pl.kernel(
    _deg_body,
    out_type=jax.ShapeDtypeStruct((NC, NP, L), jnp.float32),
    mesh=_mesh(),
    compiler_params=pltpu.CompilerParams(use_tc_tiling_on_sc=False),
    scratch_types=[
        pltpu.VMEM((D_K, 128), jnp.int32),
        pltpu.VMEM((128,), jnp.float32),
        pltpu.VMEM((ZCH,), jnp.float32),
        pltpu.VMEM((ZCH, L), jnp.float32),
        pltpu.VMEM_SHARED((NP,), jnp.float32),
    ],
  )


def _pool_body(a4, batchr, scale, shift, out, idx_b, rowbuf, onesbuf, zbuf,
               sb, cb, svb, shb, obuf, accp, acccnt):
    c = lax.axis_index("c")
    s = lax.axis_index("s")
    _fill(zbuf, GP, 0.0)
    _fill(onesbuf, 128, 1.0)

    @pl.when(s == 0)
    def _():
        pltpu.sync_copy(zbuf.at[pl.ds(0, GP)], accp)
        pltpu.sync_copy(zbuf.at[pl.ds(0, GP)], acccnt)
    plsc.subcore_barrier()

    t0 = s * (NP // NS)
    pltpu.sync_copy(batchr, idx_b)

    def unit(u, _):
        pltpu.sync_copy(a4.at[c, pl.ds(t0 + u * 128, 128)], rowbuf)
        pltpu.sync_copy(rowbuf, accp.at[idx_b.at[s * P_U + u]], add=True)
        pltpu.sync_copy(onesbuf, acccnt.at[idx_b.at[s * P_U + u]], add=True)
        return 0
    lax.fori_loop(0, P_U, unit, 0)
    plsc.subcore_barrier()

    @pl.when(s == 0)
    def _():
        pltpu.sync_copy(accp.at[pl.ds(0, G)], sb)
        pltpu.sync_copy(acccnt.at[pl.ds(0, G)], cb)
        pltpu.sync_copy(scale.at[pl.ds(c * 8, 8)], svb)
        pltpu.sync_copy(shift.at[pl.ds(c * 8, 8)], shb)

        def row(i, _):
            m = sb[i] / jnp.maximum(cb[i], 1.0)
            obuf[i] = m * svb[0] + shb[0]
            return 0
        lax.fori_loop(0, G, row, 0)
        pltpu.sync_copy(obuf, out.at[c])


@functools.cache
def _pool():
  return pl.kernel(
    _pool_body,
    out_type=jax.ShapeDtypeStruct((NC, G, L), jnp.float32),
    mesh=_mesh(),
    compiler_params=pltpu.CompilerParams(use_tc_tiling_on_sc=False),
    scratch_types=[
        pltpu.VMEM((NP // 128, 128), jnp.int32),
        pltpu.VMEM((128, L), jnp.float32),
        pltpu.VMEM((128, L), jnp.float32),
        pltpu.VMEM((GP, L), jnp.float32),
        pltpu.VMEM((G, L), jnp.float32),
        pltpu.VMEM((G, L), jnp.float32),
        pltpu.VMEM((8, L), jnp.float32),
        pltpu.VMEM((8, L), jnp.float32),
        pltpu.VMEM((G, L), jnp.float32),
        pltpu.VMEM_SHARED((GP, L), jnp.float32),
        pltpu.VMEM_SHARED((GP, L), jnp.float32),
    ],
  )


BN_ROWS = 1568
GRID = NP // BN_ROWS


def _stats_kernel(x_ref, o_ref):
    i = pl.program_id(0)
    x = x_ref[...]
    sm = jnp.sum(x, axis=0)
    sq = jnp.sum(x * x, axis=0)
    blk = jnp.concatenate(
        [sm[None, :], sq[None, :], jnp.zeros((6, x.shape[1]), jnp.float32)], axis=0)

    @pl.when(i == 0)
    def _():
        o_ref[...] = jnp.zeros_like(o_ref)
    o_ref[...] += blk


def _stats(x):
    d = x.shape[1]
    return pl.pallas_call(
        _stats_kernel,
        grid=(GRID,),
        in_specs=[pl.BlockSpec((BN_ROWS, d), lambda i: (i, 0))],
        out_specs=pl.BlockSpec((8, d), lambda i: (0, 0)),
        out_shape=jax.ShapeDtypeStruct((8, d), jnp.float32),
    )(x)


def _dinv_kernel(deg_ref, o_ref):
    o_ref[...] = lax.rsqrt(deg_ref[0] + deg_ref[1] + 1.0)


def _dinv16(degp):
    return pl.pallas_call(
        _dinv_kernel,
        grid=(GRID,),
        in_specs=[pl.BlockSpec((NC, BN_ROWS, L), lambda i: (0, i, 0))],
        out_specs=pl.BlockSpec((BN_ROWS, L), lambda i: (i, 0)),
        out_shape=jax.ShapeDtypeStruct((NP, L), jnp.float32),
    )(degp)


def _hprime_kernel(t_ref, dv_ref, w_ref, b_ref, o_ref):
    dv = dv_ref[...]
    h = jnp.dot(t_ref[...], w_ref[...], preferred_element_type=jnp.float32,
                precision=lax.Precision.HIGHEST)
    h = h + b_ref[...]
    h = h * jnp.concatenate([dv, dv], axis=1)
    o_ref[0] = h[:, :L]
    o_ref[1] = h[:, L:]


def _hprime(t, dinv, wf, bf):
    din = t.shape[1]
    return pl.pallas_call(
        _hprime_kernel,
        grid=(GRID,),
        in_specs=[
            pl.BlockSpec((BN_ROWS, din), lambda i: (i, 0)),
            pl.BlockSpec((BN_ROWS, L), lambda i: (i, 0)),
            pl.BlockSpec((din, HID), lambda i: (0, 0)),
            pl.BlockSpec((1, HID), lambda i: (0, 0)),
        ],
        out_specs=pl.BlockSpec((NC, BN_ROWS, L), lambda i: (0, i, 0)),
        out_shape=jax.ShapeDtypeStruct((NC, NP, L), jnp.float32),
    )(t, dinv, wf, bf)


def _act_kernel(split_out, s_ref, h_ref, dv_ref, c_ref, o_ref, st_ref):
    i = pl.program_id(0)
    dv = dv_ref[...]
    dv2 = jnp.concatenate([dv, dv], axis=1)
    agg = jnp.concatenate([s_ref[0] + h_ref[0], s_ref[1] + h_ref[1]], axis=1)
    t = jnp.maximum(dv2 * agg + c_ref[...], 0.0)
    rows = lax.broadcasted_iota(jnp.int32, t.shape, 0)
    t = jnp.where(rows < (N - i * BN_ROWS), t, 0.0)
    if split_out:
        o_ref[0] = t[:, :L]
        o_ref[1] = t[:, L:]
    else:
        o_ref[...] = t
    sm = jnp.sum(t, axis=0)
    sq = jnp.sum(t * t, axis=0)
    blk = jnp.concatenate(
        [sm[None, :], sq[None, :], jnp.zeros((6, HID), jnp.float32)], axis=0)

    @pl.when(i == 0)
    def _():
        st_ref[...] = jnp.zeros_like(st_ref)
    st_ref[...] += blk


def _act(S, hp, dinv, cc, split_out):
    if split_out:
        ospec = pl.BlockSpec((NC, BN_ROWS, L), lambda i: (0, i, 0))
        oshape = jax.ShapeDtypeStruct((NC, NP, L), jnp.float32)
    else:
        ospec = pl.BlockSpec((BN_ROWS, HID), lambda i: (i, 0))
        oshape = jax.ShapeDtypeStruct((NP, HID), jnp.float32)
    return pl.pallas_call(
        functools.partial(_act_kernel, split_out),
        grid=(GRID,),
        in_specs=[
            pl.BlockSpec((NC, BN_ROWS, L), lambda i: (0, i, 0)),
            pl.BlockSpec((NC, BN_ROWS, L), lambda i: (0, i, 0)),
            pl.BlockSpec((BN_ROWS, L), lambda i: (i, 0)),
            pl.BlockSpec((1, HID), lambda i: (0, 0)),
        ],
        out_specs=[ospec, pl.BlockSpec((8, HID), lambda i: (0, 0))],
        out_shape=[oshape, jax.ShapeDtypeStruct((8, HID), jnp.float32)],
    )(S, hp, dinv, cc)


def _finalize(st, g, be):
    mu = st[0] / N
    var = st[1] / N - mu * mu
    k = g * lax.rsqrt(var + EPS)
    return k, be - mu * k


def kernel(pos, edge_index, batch, W1, c1, W2, c2, W3, c3, W4, c4,
           g0, be0, g1, be1, g2, be2, g3, be3, g4, be4):
    src = edge_index[0]
    dst = edge_index[1]
    src_p = jnp.concatenate(
        [src, jnp.zeros((EP - E,), jnp.int32)]).reshape(RT, 128)
    dst_p = jnp.concatenate(
        [dst, jnp.full((EP - E,), DUMP, jnp.int32)]).reshape(RT, 128)
    batch_p = jnp.concatenate(
        [batch, jnp.full((NP - N,), G, jnp.int32)]).reshape(NP // 128, 128)
    pos_p = jnp.concatenate(
        [pos, jnp.zeros((NP - N, IN_CH), jnp.float32)], axis=0)

    degp = _deg_count()(dst_p)
    dinv = _dinv16(degp)

    stp = _stats(pos_p)
    k0, b0 = _finalize(stp, g0, be0)

    t = pos_p
    Ws = [W1, W2, W3, W4]
    cs = [c1, c2, c3, c4]
    gs = [g1, g2, g3, g4]
    bes = [be1, be2, be3, be4]
    kprev, bprev = k0, b0
    st = None
    for i in range(4):
        wf = kprev[:, None] * Ws[i]
        bf = jnp.dot(bprev, Ws[i],
                     precision=lax.Precision.HIGHEST).reshape(1, HID)
        hp = _hprime(t, dinv, wf, bf)
        S = _seg_sum()(hp, src_p, dst_p)
        t, st = _act(S, hp, dinv, cs[i].reshape(1, HID), split_out=(i == 3))
        if i < 3:
            kprev, bprev = _finalize(st, gs[i], bes[i])

    k4, b4 = _finalize(st, g4, be4)
    z7 = jnp.zeros((7, L), jnp.float32)
    scale = jnp.concatenate(
        [k4[:L].reshape(1, L), z7, k4[L:].reshape(1, L), z7], axis=0)
    shift = jnp.concatenate(
        [b4[:L].reshape(1, L), z7, b4[L:].reshape(1, L), z7], axis=0)
    pooled = _pool()(t, batch_p, scale, shift)
    return jnp.concatenate([pooled[0], pooled[1]], axis=1)


# speedup vs baseline: 20.3696x; 1.7738x over previous
import functools

import jax
import jax.numpy as jnp
from jax import lax
from jax.experimental import pallas as pl
from jax.experimental.pallas import tpu as pltpu
from jax.experimental.pallas import tpu_sc as plsc

N = 100000
E = 1600000
IN_CH = 13
HID = 32
G = 64
EPS = 1e-5

NC = 2
NS = 16
L = 16

NP = 100352
DUMP = 100000
EP = 1605632
RT = EP // 128

B_ROWS = RT // NS
B_K = 56
B_MC = B_ROWS // B_K
NBUF = 7
B_NG = B_K // NBUF
D_ROWS = RT // (NC * NS)
D_K = 56
D_MC = D_ROWS // D_K
ZCH = 784
ZN = NP // NS // ZCH
P_U = NP // NS // 128
GP = 72

@functools.cache
def _mesh():
    return plsc.VectorSubcoreMesh(core_axis_name="c", subcore_axis_name="s",
                                  num_cores=NC, num_subcores=NS)


def _fill(ref, rows, value):
    def body(i, _):
        ref[i] = jnp.full((L,), value, jnp.float32)
        return 0
    lax.fori_loop(0, rows, body, 0)


def _zero_acc(acc, zbuf, s):
    _fill(zbuf, ZCH, 0.0)
    t0 = s * (NP // NS)
    for k in range(ZN):
        pltpu.sync_copy(zbuf.at[pl.ds(0, ZCH)], acc.at[pl.ds(t0 + k * ZCH, ZCH)])


def _write_acc(acc, out, c, s):
    t0 = s * (NP // NS)
    for k in range(ZN):
        pltpu.sync_copy(acc.at[pl.ds(t0 + k * ZCH, ZCH)],
                        out.at[c, pl.ds(t0 + k * ZCH, ZCH)])


def _seg_body(h2, srcr, dstr, out, idx_s, idx_d, rowbuf, acc, gsem, ssem):
    c = lax.axis_index("c")
    s = lax.axis_index("s")
    _fill(rowbuf.at[0], 128, 0.0)
    t0 = s * (NP // NS)

    def zrow(k, _):
        pltpu.sync_copy(rowbuf.at[0], acc.at[pl.ds(t0 + k * 128, 128)])
        return 0
    lax.fori_loop(0, NP // NS // 128, zrow, 0)
    plsc.subcore_barrier()
    tbl = h2.at[c]

    def macro(m, _):
        base = s * B_ROWS + m * B_K
        pltpu.sync_copy(srcr.at[pl.ds(base, B_K)], idx_s)
        pltpu.sync_copy(dstr.at[pl.ds(base, B_K)], idx_d)
        for b in range(NBUF):
            pltpu.async_copy(tbl.at[idx_s.at[b]], rowbuf.at[b], gsem.at[b])

        def group(g, _):
            for b in range(NBUF):
                r = g * NBUF + b
                pltpu.make_async_copy(
                    tbl.at[idx_s.at[r]], rowbuf.at[b], gsem.at[b]).wait()
                pltpu.async_copy(rowbuf.at[b], acc.at[idx_d.at[r]], ssem.at[b],
                                 add=True)
            for b in range(NBUF):
                r = g * NBUF + b
                pltpu.make_async_copy(
                    rowbuf.at[b], acc.at[idx_d.at[r]], ssem.at[b]).wait()

                @pl.when(g < B_NG - 1)
                def _():
                    pltpu.async_copy(tbl.at[idx_s.at[r + NBUF]], rowbuf.at[b],
                                     gsem.at[b])
            return 0
        lax.fori_loop(0, B_NG, group, 0)
        return 0
    lax.fori_loop(0, B_MC, macro, 0)
    plsc.subcore_barrier()
    _write_acc(acc, out, c, s)


@functools.cache
def _seg_sum():
  return pl.kernel(
    _seg_body,
    out_type=jax.ShapeDtypeStruct((NC, NP, L), jnp.float32),
    mesh=_mesh(),
    compiler_params=pltpu.CompilerParams(use_tc_tiling_on_sc=False),
    scratch_types=[
        pltpu.VMEM((B_K, 128), jnp.int32),
        pltpu.VMEM((B_K, 128), jnp.int32),
        pltpu.VMEM((NBUF, 128, L), jnp.float32),
        pltpu.VMEM_SHARED((NP, L), jnp.float32),
        pltpu.SemaphoreType.DMA((NBUF,)),
        pltpu.SemaphoreType.DMA((NBUF,)),
    ],
  )


def _fill1(ref, words, value):
    def body(i, _):
        ref[pl.ds(i * L, L)] = jnp.full((L,), value, jnp.float32)
        return 0
    lax.fori_loop(0, words // L, body, 0)


def _deg_body(dstr, out, idx_d, onesb, wbuf, ebuf, acc1):
    c = lax.axis_index("c")
    s = lax.axis_index("s")
    _fill1(wbuf, ZCH, 0.0)
    t0 = s * (NP // NS)
    for k in range(ZN):
        pltpu.sync_copy(wbuf.at[pl.ds(0, ZCH)], acc1.at[pl.ds(t0 + k * ZCH, ZCH)])
    _fill1(onesb, 128, 1.0)
    plsc.subcore_barrier()

    def macro(m, _):
        base = (c * NS + s) * D_ROWS + m * D_K
        pltpu.sync_copy(dstr.at[pl.ds(base, D_K)], idx_d)

        def inner(j, _):
            pltpu.sync_copy(onesb, acc1.at[idx_d.at[j]], add=True)
            return 0
        lax.fori_loop(0, D_K, inner, 0)
        return 0
    lax.fori_loop(0, D_MC, macro, 0)
    plsc.subcore_barrier()
    for k in range(ZN):
        r0 = t0 + k * ZCH
        pltpu.sync_copy(acc1.at[pl.ds(r0, ZCH)], wbuf)

        def expand(i, _):
            v = wbuf[pl.ds(i * L, L)]
            for kk in range(L):
                ebuf[i * L + kk] = jnp.broadcast_to(v[kk], (L,))
            return 0
        lax.fori_loop(0, ZCH // L, expand, 0)
        pltpu.sync_copy(ebuf, out.at[c, pl.ds(r0, ZCH)])


@functools.cache
def _deg_count():
  return pl.kernel(
    _deg_body,
    out_type=jax.ShapeDtypeStruct((NC, NP, L), jnp.float32),
    mesh=_mesh(),
    compiler_params=pltpu.CompilerParams(use_tc_tiling_on_sc=False),
    scratch_types=[
        pltpu.VMEM((D_K, 128), jnp.int32),
        pltpu.VMEM((128,), jnp.float32),
        pltpu.VMEM((ZCH,), jnp.float32),
        pltpu.VMEM((ZCH, L), jnp.float32),
        pltpu.VMEM_SHARED((NP,), jnp.float32),
    ],
  )


def _pool_body(a4, batchr, scale, shift, out, idx_b, rowbuf, onesbuf, zbuf,
               sb, cb, svb, shb, obuf, accp, acccnt):
    c = lax.axis_index("c")
    s = lax.axis_index("s")
    _fill(zbuf, GP, 0.0)
    _fill(onesbuf, 128, 1.0)

    @pl.when(s == 0)
    def _():
        pltpu.sync_copy(zbuf.at[pl.ds(0, GP)], accp)
        pltpu.sync_copy(zbuf.at[pl.ds(0, GP)], acccnt)
    plsc.subcore_barrier()

    t0 = s * (NP // NS)
    pltpu.sync_copy(batchr, idx_b)

    def unit(u, _):
        pltpu.sync_copy(a4.at[c, pl.ds(t0 + u * 128, 128)], rowbuf)
        pltpu.sync_copy(rowbuf, accp.at[idx_b.at[s * P_U + u]], add=True)
        pltpu.sync_copy(onesbuf, acccnt.at[idx_b.at[s * P_U + u]], add=True)
        return 0
    lax.fori_loop(0, P_U, unit, 0)
    plsc.subcore_barrier()

    @pl.when(s == 0)
    def _():
        pltpu.sync_copy(accp.at[pl.ds(0, G)], sb)
        pltpu.sync_copy(acccnt.at[pl.ds(0, G)], cb)
        pltpu.sync_copy(scale.at[pl.ds(c * 8, 8)], svb)
        pltpu.sync_copy(shift.at[pl.ds(c * 8, 8)], shb)

        def row(i, _):
            m = sb[i] / jnp.maximum(cb[i], 1.0)
            obuf[i] = m * svb[0] + shb[0]
            return 0
        lax.fori_loop(0, G, row, 0)
        pltpu.sync_copy(obuf, out.at[c])


@functools.cache
def _pool():
  return pl.kernel(
    _pool_body,
    out_type=jax.ShapeDtypeStruct((NC, G, L), jnp.float32),
    mesh=_mesh(),
    compiler_params=pltpu.CompilerParams(use_tc_tiling_on_sc=False),
    scratch_types=[
        pltpu.VMEM((NP // 128, 128), jnp.int32),
        pltpu.VMEM((128, L), jnp.float32),
        pltpu.VMEM((128, L), jnp.float32),
        pltpu.VMEM((GP, L), jnp.float32),
        pltpu.VMEM((G, L), jnp.float32),
        pltpu.VMEM((G, L), jnp.float32),
        pltpu.VMEM((8, L), jnp.float32),
        pltpu.VMEM((8, L), jnp.float32),
        pltpu.VMEM((G, L), jnp.float32),
        pltpu.VMEM_SHARED((GP, L), jnp.float32),
        pltpu.VMEM_SHARED((GP, L), jnp.float32),
    ],
  )


BN_ROWS = 1568
GRID = NP // BN_ROWS


def _stats_kernel(x_ref, o_ref):
    i = pl.program_id(0)
    x = x_ref[...]
    sm = jnp.sum(x, axis=0)
    sq = jnp.sum(x * x, axis=0)
    blk = jnp.concatenate(
        [sm[None, :], sq[None, :], jnp.zeros((6, x.shape[1]), jnp.float32)], axis=0)

    @pl.when(i == 0)
    def _():
        o_ref[...] = jnp.zeros_like(o_ref)
    o_ref[...] += blk


def _stats(x):
    d = x.shape[1]
    return pl.pallas_call(
        _stats_kernel,
        grid=(GRID,),
        in_specs=[pl.BlockSpec((BN_ROWS, d), lambda i: (i, 0))],
        out_specs=pl.BlockSpec((8, d), lambda i: (0, 0)),
        out_shape=jax.ShapeDtypeStruct((8, d), jnp.float32),
    )(x)


def _dinv_kernel(deg_ref, o_ref):
    o_ref[...] = lax.rsqrt(deg_ref[0] + deg_ref[1] + 1.0)


def _dinv16(degp):
    return pl.pallas_call(
        _dinv_kernel,
        grid=(GRID,),
        in_specs=[pl.BlockSpec((NC, BN_ROWS, L), lambda i: (0, i, 0))],
        out_specs=pl.BlockSpec((BN_ROWS, L), lambda i: (i, 0)),
        out_shape=jax.ShapeDtypeStruct((NP, L), jnp.float32),
    )(degp)


def _hprime_kernel(t_ref, dv_ref, w_ref, b_ref, o_ref):
    dv = dv_ref[...]
    h = jnp.dot(t_ref[...], w_ref[...], preferred_element_type=jnp.float32,
                precision=lax.Precision.HIGHEST)
    h = h + b_ref[...]
    h = h * jnp.concatenate([dv, dv], axis=1)
    o_ref[0] = h[:, :L]
    o_ref[1] = h[:, L:]


def _hprime(t, dinv, wf, bf):
    din = t.shape[1]
    return pl.pallas_call(
        _hprime_kernel,
        grid=(GRID,),
        in_specs=[
            pl.BlockSpec((BN_ROWS, din), lambda i: (i, 0)),
            pl.BlockSpec((BN_ROWS, L), lambda i: (i, 0)),
            pl.BlockSpec((din, HID), lambda i: (0, 0)),
            pl.BlockSpec((1, HID), lambda i: (0, 0)),
        ],
        out_specs=pl.BlockSpec((NC, BN_ROWS, L), lambda i: (0, i, 0)),
        out_shape=jax.ShapeDtypeStruct((NC, NP, L), jnp.float32),
    )(t, dinv, wf, bf)


def _act_kernel(split_out, s_ref, h_ref, dv_ref, c_ref, o_ref, st_ref):
    i = pl.program_id(0)
    dv = dv_ref[...]
    dv2 = jnp.concatenate([dv, dv], axis=1)
    agg = jnp.concatenate([s_ref[0] + h_ref[0], s_ref[1] + h_ref[1]], axis=1)
    t = jnp.maximum(dv2 * agg + c_ref[...], 0.0)
    rows = lax.broadcasted_iota(jnp.int32, t.shape, 0)
    t = jnp.where(rows < (N - i * BN_ROWS), t, 0.0)
    if split_out:
        o_ref[0] = t[:, :L]
        o_ref[1] = t[:, L:]
    else:
        o_ref[...] = t
    sm = jnp.sum(t, axis=0)
    sq = jnp.sum(t * t, axis=0)
    blk = jnp.concatenate(
        [sm[None, :], sq[None, :], jnp.zeros((6, HID), jnp.float32)], axis=0)

    @pl.when(i == 0)
    def _():
        st_ref[...] = jnp.zeros_like(st_ref)
    st_ref[...] += blk


def _act(S, hp, dinv, cc, split_out):
    if split_out:
        ospec = pl.BlockSpec((NC, BN_ROWS, L), lambda i: (0, i, 0))
        oshape = jax.ShapeDtypeStruct((NC, NP, L), jnp.float32)
    else:
        ospec = pl.BlockSpec((BN_ROWS, HID), lambda i: (i, 0))
        oshape = jax.ShapeDtypeStruct((NP, HID), jnp.float32)
    return pl.pallas_call(
        functools.partial(_act_kernel, split_out),
        grid=(GRID,),
        in_specs=[
            pl.BlockSpec((NC, BN_ROWS, L), lambda i: (0, i, 0)),
            pl.BlockSpec((NC, BN_ROWS, L), lambda i: (0, i, 0)),
            pl.BlockSpec((BN_ROWS, L), lambda i: (i, 0)),
            pl.BlockSpec((1, HID), lambda i: (0, 0)),
        ],
        out_specs=[ospec, pl.BlockSpec((8, HID), lambda i: (0, 0))],
        out_shape=[oshape, jax.ShapeDtypeStruct((8, HID), jnp.float32)],
    )(S, hp, dinv, cc)


def _finalize(st, g, be):
    mu = st[0] / N
    var = st[1] / N - mu * mu
    k = g * lax.rsqrt(var + EPS)
    return k, be - mu * k


def kernel(pos, edge_index, batch, W1, c1, W2, c2, W3, c3, W4, c4,
           g0, be0, g1, be1, g2, be2, g3, be3, g4, be4):
    src = edge_index[0]
    dst = edge_index[1]
    src_p = jnp.concatenate(
        [src, jnp.zeros((EP - E,), jnp.int32)]).reshape(RT, 128)
    dst_p = jnp.concatenate(
        [dst, jnp.full((EP - E,), DUMP, jnp.int32)]).reshape(RT, 128)
    batch_p = jnp.concatenate(
        [batch, jnp.full((NP - N,), G, jnp.int32)]).reshape(NP // 128, 128)
    pos_p = jnp.concatenate(
        [pos, jnp.zeros((NP - N, IN_CH), jnp.float32)], axis=0)

    degp = _deg_count()(dst_p)
    dinv = _dinv16(degp)

    stp = _stats(pos_p)
    k0, b0 = _finalize(stp, g0, be0)

    t = pos_p
    Ws = [W1, W2, W3, W4]
    cs = [c1, c2, c3, c4]
    gs = [g1, g2, g3, g4]
    bes = [be1, be2, be3, be4]
    kprev, bprev = k0, b0
    st = None
    for i in range(4):
        wf = kprev[:, None] * Ws[i]
        bf = jnp.dot(bprev, Ws[i],
                     precision=lax.Precision.HIGHEST).reshape(1, HID)
        hp = _hprime(t, dinv, wf, bf)
        S = _seg_sum()(hp, src_p, dst_p)
        t, st = _act(S, hp, dinv, cs[i].reshape(1, HID), split_out=(i == 3))
        if i < 3:
            kprev, bprev = _finalize(st, gs[i], bes[i])

    k4, b4 = _finalize(st, g4, be4)
    z7 = jnp.zeros((7, L), jnp.float32)
    scale = jnp.concatenate(
        [k4[:L].reshape(1, L), z7, k4[L:].reshape(1, L), z7], axis=0)
    shift = jnp.concatenate(
        [b4[:L].reshape(1, L), z7, b4[L:].reshape(1, L), z7], axis=0)
    pooled = _pool()(t, batch_p, scale, shift)
    return jnp.concatenate([pooled[0], pooled[1]], axis=1)


# speedup vs baseline: 34.4358x; 1.6906x over previous
import functools

import jax
import jax.numpy as jnp
from jax import lax
from jax.experimental import pallas as pl
from jax.experimental.pallas import tpu as pltpu
from jax.experimental.pallas import tpu_sc as plsc

N = 100000
E = 1600000
IN_CH = 13
HID = 32
G = 64
EPS = 1e-5

NC = 2
NS = 16
L = 16

NP = 100352
DUMP = 100000
EP = 1605632
RT = EP // 128

B_ROWS = RT // NS
B_K = 56
B_MC = B_ROWS // B_K
NBUF = 7
B_NG = B_K // NBUF
D_ROWS = RT // (NC * NS)
D_K = 56
D_MC = D_ROWS // D_K
ZCH = 784
ZN = NP // NS // ZCH
P_U = NP // NS // 128
GP = 72

@functools.cache
def _mesh():
    return plsc.VectorSubcoreMesh(core_axis_name="c", subcore_axis_name="s",
                                  num_cores=NC, num_subcores=NS)


def _fill(ref, rows, value):
    def body(i, _):
        ref[i] = jnp.full((L,), value, jnp.float32)
        return 0
    lax.fori_loop(0, rows, body, 0)


def _zero_acc(acc, zbuf, s):
    _fill(zbuf, ZCH, 0.0)
    t0 = s * (NP // NS)
    for k in range(ZN):
        pltpu.sync_copy(zbuf.at[pl.ds(0, ZCH)], acc.at[pl.ds(t0 + k * ZCH, ZCH)])


def _write_acc(acc, out, c, s):
    t0 = s * (NP // NS)
    for k in range(ZN):
        pltpu.sync_copy(acc.at[pl.ds(t0 + k * ZCH, ZCH)],
                        out.at[c, pl.ds(t0 + k * ZCH, ZCH)])


def _seg_body(h2, srcr, dstr, out, idx_s, idx_d, rowbuf, acc, gsem, ssem):
    c = lax.axis_index("c")
    s = lax.axis_index("s")
    _fill(rowbuf.at[0], 128, 0.0)
    t0 = s * (NP // NS)

    def zrow(k, _):
        pltpu.sync_copy(rowbuf.at[0], acc.at[pl.ds(t0 + k * 128, 128)])
        return 0
    lax.fori_loop(0, NP // NS // 128, zrow, 0)
    plsc.subcore_barrier()
    tbl = h2.at[c]

    def macro(m, _):
        base = s * B_ROWS + m * B_K
        pltpu.sync_copy(srcr.at[pl.ds(base, B_K)], idx_s)
        pltpu.sync_copy(dstr.at[pl.ds(base, B_K)], idx_d)
        for b in range(NBUF):
            pltpu.async_copy(tbl.at[idx_s.at[b]], rowbuf.at[b], gsem.at[b])

        def group(g, _):
            for b in range(NBUF):
                r = g * NBUF + b
                pltpu.make_async_copy(
                    tbl.at[idx_s.at[r]], rowbuf.at[b], gsem.at[b]).wait()
                pltpu.async_copy(rowbuf.at[b], acc.at[idx_d.at[r]], ssem.at[b],
                                 add=True)
            for b in range(NBUF):
                r = g * NBUF + b
                pltpu.make_async_copy(
                    rowbuf.at[b], acc.at[idx_d.at[r]], ssem.at[b]).wait()

                @pl.when(g < B_NG - 1)
                def _():
                    pltpu.async_copy(tbl.at[idx_s.at[r + NBUF]], rowbuf.at[b],
                                     gsem.at[b])
            return 0
        lax.fori_loop(0, B_NG, group, 0)
        return 0
    lax.fori_loop(0, B_MC, macro, 0)
    plsc.subcore_barrier()
    _write_acc(acc, out, c, s)


@functools.cache
def _seg_sum():
  return pl.kernel(
    _seg_body,
    out_type=jax.ShapeDtypeStruct((NC, NP, L), jnp.float32),
    mesh=_mesh(),
    compiler_params=pltpu.CompilerParams(use_tc_tiling_on_sc=False),
    scratch_types=[
        pltpu.VMEM((B_K, 128), jnp.int32),
        pltpu.VMEM((B_K, 128), jnp.int32),
        pltpu.VMEM((NBUF, 128, L), jnp.float32),
        pltpu.VMEM_SHARED((NP, L), jnp.float32),
        pltpu.SemaphoreType.DMA((NBUF,)),
        pltpu.SemaphoreType.DMA((NBUF,)),
    ],
  )


def _fill1(ref, words, value):
    def body(i, _):
        ref[pl.ds(i * L, L)] = jnp.full((L,), value, jnp.float32)
        return 0
    lax.fori_loop(0, words // L, body, 0)


def _deg_body(dstr, out, idx_d, onesb, wbuf, ebuf, acc1):
    c = lax.axis_index("c")
    s = lax.axis_index("s")
    _fill1(wbuf, ZCH, 0.0)
    t0 = s * (NP // NS)
    for k in range(ZN):
        pltpu.sync_copy(wbuf.at[pl.ds(0, ZCH)], acc1.at[pl.ds(t0 + k * ZCH, ZCH)])
    _fill1(onesb, 128, 1.0)
    plsc.subcore_barrier()

    def macro(m, _):
        base = (c * NS + s) * D_ROWS + m * D_K
        pltpu.sync_copy(dstr.at[pl.ds(base, D_K)], idx_d)

        def inner(j, _):
            pltpu.sync_copy(onesb, acc1.at[idx_d.at[j]], add=True)
            return 0
        lax.fori_loop(0, D_K, inner, 0)
        return 0
    lax.fori_loop(0, D_MC, macro, 0)
    plsc.subcore_barrier()
    for k in range(ZN):
        r0 = t0 + k * ZCH
        pltpu.sync_copy(acc1.at[pl.ds(r0, ZCH)], wbuf)

        def expand(i, _):
            v = wbuf[pl.ds(i * L, L)]
            for kk in range(L):
                ebuf[i * L + kk] = jnp.broadcast_to(v[kk], (L,))
            return 0
        lax.fori_loop(0, ZCH // L, expand, 0)
        pltpu.sync_copy(ebuf, out.at[c, pl.ds(r0, ZCH)])


@functools.cache
def _deg_count():
  return pl.kernel(
    _deg_body,
    out_type=jax.ShapeDtypeStruct((NC, NP, L), jnp.float32),
    mesh=_mesh(),
    compiler_params=pltpu.CompilerParams(use_tc_tiling_on_sc=False),
    scratch_types=[
        pltpu.VMEM((D_K, 128), jnp.int32),
        pltpu.VMEM((128,), jnp.float32),
        pltpu.VMEM((ZCH,), jnp.float32),
        pltpu.VMEM((ZCH, L), jnp.float32),
        pltpu.VMEM_SHARED((NP,), jnp.float32),
    ],
  )


def _pool_body(a4, batchr, scale, shift, out, idx_b, rowbuf, onesbuf, zbuf,
               sb, cb, svb, shb, obuf, accp, acccnt):
    c = lax.axis_index("c")
    s = lax.axis_index("s")
    _fill(zbuf, GP, 0.0)
    _fill(onesbuf, 128, 1.0)

    @pl.when(s == 0)
    def _():
        pltpu.sync_copy(zbuf.at[pl.ds(0, GP)], accp)
        pltpu.sync_copy(zbuf.at[pl.ds(0, GP)], acccnt)
    plsc.subcore_barrier()

    t0 = s * (NP // NS)
    pltpu.sync_copy(batchr, idx_b)

    def unit(u, _):
        pltpu.sync_copy(a4.at[c, pl.ds(t0 + u * 128, 128)], rowbuf)
        pltpu.sync_copy(rowbuf, accp.at[idx_b.at[s * P_U + u]], add=True)
        pltpu.sync_copy(onesbuf, acccnt.at[idx_b.at[s * P_U + u]], add=True)
        return 0
    lax.fori_loop(0, P_U, unit, 0)
    plsc.subcore_barrier()

    @pl.when(s == 0)
    def _():
        pltpu.sync_copy(accp.at[pl.ds(0, G)], sb)
        pltpu.sync_copy(acccnt.at[pl.ds(0, G)], cb)
        pltpu.sync_copy(scale.at[pl.ds(c * 8, 8)], svb)
        pltpu.sync_copy(shift.at[pl.ds(c * 8, 8)], shb)

        def row(i, _):
            m = sb[i] / jnp.maximum(cb[i], 1.0)
            obuf[i] = m * svb[0] + shb[0]
            return 0
        lax.fori_loop(0, G, row, 0)
        pltpu.sync_copy(obuf, out.at[c])


@functools.cache
def _pool():
  return pl.kernel(
    _pool_body,
    out_type=jax.ShapeDtypeStruct((NC, G, L), jnp.float32),
    mesh=_mesh(),
    compiler_params=pltpu.CompilerParams(use_tc_tiling_on_sc=False),
    scratch_types=[
        pltpu.VMEM((NP // 128, 128), jnp.int32),
        pltpu.VMEM((128, L), jnp.float32),
        pltpu.VMEM((128, L), jnp.float32),
        pltpu.VMEM((GP, L), jnp.float32),
        pltpu.VMEM((G, L), jnp.float32),
        pltpu.VMEM((G, L), jnp.float32),
        pltpu.VMEM((8, L), jnp.float32),
        pltpu.VMEM((8, L), jnp.float32),
        pltpu.VMEM((G, L), jnp.float32),
        pltpu.VMEM_SHARED((GP, L), jnp.float32),
        pltpu.VMEM_SHARED((GP, L), jnp.float32),
    ],
  )


NF = NP // 8
GRID = 56
BNF = NF // GRID


def _stats_kernel(x_ref, o_ref):
    i = pl.program_id(0)
    x = x_ref[...]
    blk = jnp.concatenate(
        [jnp.sum(x, axis=0)[None, :], jnp.sum(x * x, axis=0)[None, :],
         jnp.zeros((6, 128), jnp.float32)], axis=0)

    @pl.when(i == 0)
    def _():
        o_ref[...] = jnp.zeros_like(o_ref)
    o_ref[...] += blk


def _stats(x):
    return pl.pallas_call(
        _stats_kernel,
        grid=(GRID,),
        in_specs=[pl.BlockSpec((BNF, 128), lambda i: (i, 0))],
        out_specs=pl.BlockSpec((8, 128), lambda i: (0, 0)),
        out_shape=jax.ShapeDtypeStruct((8, 128), jnp.float32),
    )(x)


def _dinv_kernel(deg_ref, o_ref):
    o_ref[...] = lax.rsqrt(deg_ref[0] + deg_ref[1] + 1.0)


def _dinv16(degf):
    return pl.pallas_call(
        _dinv_kernel,
        grid=(GRID,),
        in_specs=[pl.BlockSpec((NC, BNF, 128), lambda i: (0, i, 0))],
        out_specs=pl.BlockSpec((BNF, 128), lambda i: (i, 0)),
        out_shape=jax.ShapeDtypeStruct((NF, 128), jnp.float32),
    )(degf)


def _dot(a, b):
    return jnp.dot(a, b, preferred_element_type=jnp.float32,
                   precision=lax.Precision.HIGHEST)


def _hp1_kernel(x_ref, dv_ref, k1, k2, b_ref, o_ref):
    dv = dv_ref[...]
    x = x_ref[...]
    o_ref[0] = (_dot(x, k1[...]) + b_ref[0:1, :]) * dv
    o_ref[1] = (_dot(x, k2[...]) + b_ref[1:2, :]) * dv


def _hp_kernel(t_ref, dv_ref, kll, khl, klh, khh, b_ref, o_ref):
    dv = dv_ref[...]
    xlo = t_ref[0]
    xhi = t_ref[1]
    o_ref[0] = (_dot(xlo, kll[...]) + _dot(xhi, khl[...]) + b_ref[0:1, :]) * dv
    o_ref[1] = (_dot(xlo, klh[...]) + _dot(xhi, khh[...]) + b_ref[1:2, :]) * dv


_KSPEC = pl.BlockSpec((128, 128), lambda i: (0, 0))
_HSPEC = pl.BlockSpec((NC, BNF, 128), lambda i: (0, i, 0))
_FSPEC = pl.BlockSpec((BNF, 128), lambda i: (i, 0))
_R2SPEC = pl.BlockSpec((2, 128), lambda i: (0, 0))
_HSHAPE = jax.ShapeDtypeStruct((NC, NF, 128), jnp.float32)


def _hprime1(x, dinv, K1, K2, brep):
    return pl.pallas_call(
        _hp1_kernel, grid=(GRID,),
        in_specs=[_FSPEC, _FSPEC, _KSPEC, _KSPEC, _R2SPEC],
        out_specs=_HSPEC, out_shape=_HSHAPE,
    )(x, dinv, K1, K2, brep)


def _hprime(t, dinv, Ks, brep):
    return pl.pallas_call(
        _hp_kernel, grid=(GRID,),
        in_specs=[_HSPEC, _FSPEC, _KSPEC, _KSPEC, _KSPEC, _KSPEC, _R2SPEC],
        out_specs=_HSPEC, out_shape=_HSHAPE,
    )(t, dinv, *Ks, brep)


def _act_kernel(s_ref, h_ref, dv_ref, c_ref, o_ref, st_ref):
    i = pl.program_id(0)
    dv = dv_ref[...]
    rows = lax.broadcasted_iota(jnp.int32, (BNF, 128), 0)
    lanes = lax.broadcasted_iota(jnp.int32, (BNF, 128), 1)
    nid = (i * BNF + rows) * 8 + lanes // L
    valid = nid < N
    sums = []
    for h in range(2):
        tt = dv * (s_ref[h] + h_ref[h]) + c_ref[h:h + 1, :]
        tt = jnp.where(valid, jnp.maximum(tt, 0.0), 0.0)
        o_ref[h] = tt
        sums.append(tt)
    blk = jnp.concatenate(
        [jnp.sum(sums[0], axis=0)[None, :], jnp.sum(sums[1], axis=0)[None, :],
         jnp.sum(sums[0] * sums[0], axis=0)[None, :],
         jnp.sum(sums[1] * sums[1], axis=0)[None, :],
         jnp.zeros((4, 128), jnp.float32)], axis=0)

    @pl.when(i == 0)
    def _():
        st_ref[...] = jnp.zeros_like(st_ref)
    st_ref[...] += blk


def _act(S, hp, dinv, crep):
    return pl.pallas_call(
        _act_kernel, grid=(GRID,),
        in_specs=[_HSPEC, _HSPEC, _FSPEC, _R2SPEC],
        out_specs=[_HSPEC, pl.BlockSpec((8, 128), lambda i: (0, 0))],
        out_shape=[_HSHAPE, jax.ShapeDtypeStruct((8, 128), jnp.float32)],
    )(S, hp, dinv, crep)


_I8 = None


def _blockdiag(sub):
    return jnp.kron(jnp.eye(8, dtype=jnp.float32), sub)


def _fold16(row):
    return row.reshape(8, L).sum(axis=0)


def _finalize(sum_v, sumsq_v, g, be):
    mu = sum_v / N
    var = sumsq_v / N - mu * mu
    k = g * lax.rsqrt(var + EPS)
    return k, be - mu * k


def kernel(pos, edge_index, batch, W1, c1, W2, c2, W3, c3, W4, c4,
           g0, be0, g1, be1, g2, be2, g3, be3, g4, be4):
    src = edge_index[0]
    dst = edge_index[1]
    src_p = jnp.concatenate(
        [src, jnp.zeros((EP - E,), jnp.int32)]).reshape(RT, 128)
    dst_p = jnp.concatenate(
        [dst, jnp.full((EP - E,), DUMP, jnp.int32)]).reshape(RT, 128)
    batch_p = jnp.concatenate(
        [batch, jnp.full((NP - N,), G, jnp.int32)]).reshape(NP // 128, 128)
    pos16 = jnp.zeros((NP, L), jnp.float32).at[:N, :IN_CH].set(pos)
    pos16 = pos16.reshape(NF, 128)

    degp = _deg_count()(dst_p)
    dinv = _dinv16(degp.reshape(NC, NF, 128))

    stp = _stats(pos16)
    g016 = jnp.concatenate([g0, jnp.zeros((3,), jnp.float32)])
    be016 = jnp.concatenate([be0, jnp.zeros((3,), jnp.float32)])
    k0, b0 = _finalize(_fold16(stp[0]), _fold16(stp[1]), g016, be016)

    W1p = jnp.concatenate([W1, jnp.zeros((3, HID), jnp.float32)], axis=0)
    Ws = [W1p, W2, W3, W4]
    cs = [c1, c2, c3, c4]
    gs = [g1, g2, g3, g4]
    bes = [be1, be2, be3, be4]
    kprev, bprev = k0, b0
    t = None
    st = None
    for i in range(4):
        wf = kprev[:, None] * Ws[i]
        bf = jnp.dot(bprev, Ws[i], precision=lax.Precision.HIGHEST)
        brep = jnp.stack([jnp.tile(bf[:L], 8), jnp.tile(bf[L:], 8)])
        if i == 0:
            hp = _hprime1(pos16, dinv, _blockdiag(wf[:, :L]),
                          _blockdiag(wf[:, L:]), brep)
        else:
            Ks = [_blockdiag(wf[:L, :L]), _blockdiag(wf[L:, :L]),
                  _blockdiag(wf[:L, L:]), _blockdiag(wf[L:, L:])]
            hp = _hprime(t, dinv, Ks, brep)
        S = _seg_sum()(hp.reshape(NC, NP, L), src_p, dst_p)
        crep = jnp.stack([jnp.tile(cs[i][:L], 8), jnp.tile(cs[i][L:], 8)])
        t, st = _act(S.reshape(NC, NF, 128), hp, dinv, crep)
        if i < 3:
            sum32 = jnp.concatenate([_fold16(st[0]), _fold16(st[1])])
            sq32 = jnp.concatenate([_fold16(st[2]), _fold16(st[3])])
            kprev, bprev = _finalize(sum32, sq32, gs[i], bes[i])

    sum32 = jnp.concatenate([_fold16(st[0]), _fold16(st[1])])
    sq32 = jnp.concatenate([_fold16(st[2]), _fold16(st[3])])
    k4, b4 = _finalize(sum32, sq32, g4, be4)
    z7 = jnp.zeros((7, L), jnp.float32)
    scale = jnp.concatenate(
        [k4[:L].reshape(1, L), z7, k4[L:].reshape(1, L), z7], axis=0)
    shift = jnp.concatenate(
        [b4[:L].reshape(1, L), z7, b4[L:].reshape(1, L), z7], axis=0)
    pooled = _pool()(t.reshape(NC, NP, L), batch_p, scale, shift)
    return jnp.concatenate([pooled[0], pooled[1]], axis=1)
